# Initial kernel scaffold; baseline (speedup 1.0000x reference)
import jax
import jax.numpy as jnp
from jax.experimental import pallas as pl


def kernel(x, adjacency, W1, att_src1, att_dst1, b1, W2, att_src2, att_dst2, b2):
    raise NotImplementedError("write your pallas kernel here")


# speedup vs baseline: 73.3875x; 73.3875x over previous
import jax
import jax.numpy as jnp
from jax import lax
from jax.experimental import pallas as pl
from jax.experimental.pallas import tpu as pltpu
from jax.experimental.pallas import tpu_sc as plsc

N_NODES = 10000
N_EDGES = 320000
D_IN = 128

ROW = 128
NROWS = N_EDGES // ROW
NC, NS = 2, 16
ROWS_PER_SC = NROWS // NC
RPT_BASE = ROWS_PER_SC // NS
RPT_EXTRA = ROWS_PER_SC % NS
MAX_RPT = RPT_BASE + 1

NPAD = 10240
NPT = NPAD // NS


def _stage1_body(x_ref, wh_ref, ws_ref, wd_ref, h_ref, as_ref, ad_ref):
    x = x_ref[...]
    h_ref[...] = jnp.dot(x, wh_ref[...], preferred_element_type=jnp.float32)
    as_ref[...] = jnp.dot(x, ws_ref[...], preferred_element_type=jnp.float32)
    ad_ref[...] = jnp.dot(x, wd_ref[...], preferred_element_type=jnp.float32)


def _stage1(x, wh, ws, wd):
    blk = 1000
    return pl.pallas_call(
        _stage1_body,
        grid=(N_NODES // blk,),
        in_specs=[
            pl.BlockSpec((blk, D_IN), lambda i: (i, 0)),
            pl.BlockSpec((D_IN, 64), lambda i: (0, 0)),
            pl.BlockSpec((D_IN, 16), lambda i: (0, 0)),
            pl.BlockSpec((D_IN, 16), lambda i: (0, 0)),
        ],
        out_specs=[
            pl.BlockSpec((blk, 64), lambda i: (i, 0)),
            pl.BlockSpec((blk, 16), lambda i: (i, 0)),
            pl.BlockSpec((blk, 16), lambda i: (i, 0)),
        ],
        out_shape=[
            jax.ShapeDtypeStruct((N_NODES, 64), jnp.float32),
            jax.ShapeDtypeStruct((N_NODES, 16), jnp.float32),
            jax.ShapeDtypeStruct((N_NODES, 16), jnp.float32),
        ],
    )(x, wh, ws, wd)


def _edge1_body(h_hbm, as_hbm, ad_hbm, src_hbm, dst_hbm, outh_hbm, outp_hbm,
                idx_s, idx_d, h_buf, as_buf, ad_buf, msg_buf, p_buf,
                acch, accp, sem, sem2, sem3):
    c = lax.axis_index("c")
    s = lax.axis_index("s")

    def _zb(i, _):
        for q in range(4):
            msg_buf[i, pl.ds(q * 16, 16)] = jnp.zeros((16,), jnp.float32)
        p_buf[i, :] = jnp.zeros((16,), jnp.float32)
        return 0
    lax.fori_loop(0, ROW, _zb, 0)
    row0 = s * NPT
    for k in range(NPT // ROW):
        pltpu.sync_copy(msg_buf, acch.at[pl.ds(row0 + k * ROW, ROW)])
        pltpu.sync_copy(p_buf, accp.at[pl.ds(row0 + k * ROW, ROW)])
    plsc.subcore_barrier()

    nrows = jnp.where(s < RPT_EXTRA, RPT_BASE + 1, RPT_BASE)
    base_row = c * ROWS_PER_SC + RPT_BASE * s + jnp.minimum(s, RPT_EXTRA)

    perm = lax.rem(lax.iota(jnp.int32, 16), 8)

    def _row(j, _):
        @pl.when(j < nrows)
        def _():
            r = base_row + j
            cp_s = pltpu.async_copy(src_hbm.at[r], idx_s, sem)
            cp_d = pltpu.async_copy(dst_hbm.at[r], idx_d, sem2)
            cp_s.wait()
            cp_d.wait()
            g1 = pltpu.async_copy(h_hbm.at[idx_s], h_buf, sem)
            g2 = pltpu.async_copy(as_hbm.at[idx_s], as_buf, sem2)
            g3 = pltpu.async_copy(ad_hbm.at[idx_d], ad_buf, sem3)
            g1.wait()
            g2.wait()
            g3.wait()

            def _edge(e, _):
                a_s = as_buf[e, :]
                a_d = ad_buf[e, :]
                al = a_s + a_d
                al = jnp.maximum(al, 0.2 * al)
                p = jnp.exp(al)
                pb = jnp.take_along_axis(p, perm, axis=0)
                for q in range(4):
                    msg_buf[e, pl.ds(q * 16, 16)] = (
                        h_buf[e, pl.ds(q * 16, 16)] * pb)
                p_buf[e, :] = p
                return 0
            lax.fori_loop(0, ROW, _edge, 0)
            pltpu.sync_copy(msg_buf, acch.at[idx_d], add=True)
            pltpu.sync_copy(p_buf, accp.at[idx_d], add=True)
        return 0
    lax.fori_loop(0, MAX_RPT, _row, 0)

    plsc.subcore_barrier()
    pltpu.sync_copy(acch.at[pl.ds(row0, NPT)],
                    outh_hbm.at[c].at[pl.ds(row0, NPT)])
    pltpu.sync_copy(accp.at[pl.ds(row0, NPT)],
                    outp_hbm.at[c].at[pl.ds(row0, NPT)])


def _edge1(h, a_s, a_d, src2d, dst2d):
    mesh = plsc.VectorSubcoreMesh(core_axis_name="c", subcore_axis_name="s")
    fn = pl.kernel(
        _edge1_body,
        out_type=[
            jax.ShapeDtypeStruct((NC, NPAD, 64), jnp.float32),
            jax.ShapeDtypeStruct((NC, NPAD, 16), jnp.float32),
        ],
        mesh=mesh,
        compiler_params=pltpu.CompilerParams(use_tc_tiling_on_sc=False, needs_layout_passes=False),
        scratch_types=[
            pltpu.VMEM((ROW,), jnp.int32),
            pltpu.VMEM((ROW,), jnp.int32),
            pltpu.VMEM((ROW, 64), jnp.float32),
            pltpu.VMEM((ROW, 16), jnp.float32),
            pltpu.VMEM((ROW, 16), jnp.float32),
            pltpu.VMEM((ROW, 64), jnp.float32),
            pltpu.VMEM((ROW, 16), jnp.float32),
            pltpu.VMEM_SHARED((NPAD, 64), jnp.float32),
            pltpu.VMEM_SHARED((NPAD, 16), jnp.float32),
            pltpu.SemaphoreType.DMA,
            pltpu.SemaphoreType.DMA,
            pltpu.SemaphoreType.DMA,
        ],
    )
    return fn(h, a_s, a_d, src2d, dst2d)


def _stage3_body(ah_ref, ap_ref, b1t_ref, p2_ref, o_ref):
    num = ah_ref[0] + ah_ref[1]
    pacc = ap_ref[0] + ap_ref[1]
    den = pacc[:, 0:8] + 1e-16
    den_t = jnp.concatenate([den] * 8, axis=1)
    o = num / den_t + b1t_ref[...]
    o = jnp.where(o > 0, o, jnp.exp(o) - 1.0)
    o_ref[...] = jnp.dot(o, p2_ref[...], preferred_element_type=jnp.float32)


def _stage3(acch, accp, b1t, p2c):
    blk = 640
    return pl.pallas_call(
        _stage3_body,
        grid=(NPAD // blk,),
        in_specs=[
            pl.BlockSpec((NC, blk, 64), lambda i: (0, i, 0)),
            pl.BlockSpec((NC, blk, 16), lambda i: (0, i, 0)),
            pl.BlockSpec((1, 64), lambda i: (0, 0)),
            pl.BlockSpec((64, 4), lambda i: (0, 0)),
        ],
        out_specs=pl.BlockSpec((blk, 4), lambda i: (i, 0)),
        out_shape=jax.ShapeDtypeStruct((NPAD, 4), jnp.float32),
    )(acch, accp, b1t, p2c)


def _edge2_body(t2_hbm, src_hbm, dst_hbm, out_hbm,
                tab, idx_s, idx_d, dbuf, ibuf, zbuf, accf, sem, sem2):
    c = lax.axis_index("c")
    s = lax.axis_index("s")

    def _zb(i, _):
        zbuf[pl.ds(i * 16, 16)] = jnp.zeros((16,), jnp.float32)
        return 0
    lax.fori_loop(0, 128, _zb, 0)
    base0 = s * NPT * 16
    for k in range(NPT * 16 // 2048):
        pltpu.sync_copy(zbuf, accf.at[pl.ds(base0 + k * 2048, 2048)])
    pltpu.sync_copy(t2_hbm, tab)
    plsc.subcore_barrier()

    nrows = jnp.where(s < RPT_EXTRA, RPT_BASE + 1, RPT_BASE)
    base_row = c * ROWS_PER_SC + RPT_BASE * s + jnp.minimum(s, RPT_EXTRA)

    def _row(j, _):
        @pl.when(j < nrows)
        def _():
            r = base_row + j
            cp_s = pltpu.async_copy(src_hbm.at[r], idx_s, sem)
            cp_d = pltpu.async_copy(dst_hbm.at[r], idx_d, sem2)
            cp_s.wait()
            cp_d.wait()
            for g in range(ROW // 16):
                s16 = idx_s[pl.ds(g * 16, 16)]
                d16 = idx_d[pl.ds(g * 16, 16)]
                bs = s16 * 4
                h0 = plsc.load_gather(tab, [bs])
                h1 = plsc.load_gather(tab, [bs + 1])
                a_s = plsc.load_gather(tab, [bs + 2])
                a_d = plsc.load_gather(tab, [d16 * 4 + 3])
                al = a_s + a_d
                al = jnp.maximum(al, 0.2 * al)
                p = jnp.exp(al)
                bd = d16 * 16
                dbuf[0, pl.ds(g * 16, 16)] = p * h0
                dbuf[1, pl.ds(g * 16, 16)] = p * h1
                dbuf[2, pl.ds(g * 16, 16)] = p
                ibuf[0, pl.ds(g * 16, 16)] = bd
                ibuf[1, pl.ds(g * 16, 16)] = bd + 1
                ibuf[2, pl.ds(g * 16, 16)] = bd + 2
            for q in range(3):
                pltpu.sync_copy(dbuf.at[q], accf.at[ibuf.at[q]], add=True)
        return 0
    lax.fori_loop(0, MAX_RPT, _row, 0)

    plsc.subcore_barrier()
    pltpu.sync_copy(accf.at[pl.ds(base0, NPT * 16)],
                    out_hbm.at[c].at[pl.ds(base0, NPT * 16)])


def _edge2(t2f, src2d, dst2d):
    mesh = plsc.VectorSubcoreMesh(core_axis_name="c", subcore_axis_name="s")
    fn = pl.kernel(
        _edge2_body,
        out_type=jax.ShapeDtypeStruct((NC, NPAD * 16), jnp.float32),
        mesh=mesh,
        compiler_params=pltpu.CompilerParams(use_tc_tiling_on_sc=False, needs_layout_passes=False),
        scratch_types=[
            pltpu.VMEM((NPAD * 4,), jnp.float32),
            pltpu.VMEM((ROW,), jnp.int32),
            pltpu.VMEM((ROW,), jnp.int32),
            pltpu.VMEM((3, ROW), jnp.float32),
            pltpu.VMEM((3, ROW), jnp.int32),
            pltpu.VMEM((2048,), jnp.float32),
            pltpu.VMEM_SHARED((NPAD * 16,), jnp.float32),
            pltpu.SemaphoreType.DMA,
            pltpu.SemaphoreType.DMA,
        ],
    )
    return fn(t2f, src2d, dst2d)


def _stage5_body(a_ref, b2_ref, o_ref):
    sacc = a_ref[0] + a_ref[1]
    den = sacc[:, 2:3] + 1e-16
    o = sacc[:, 0:2] / den + b2_ref[...]
    m = jnp.max(o, axis=1, keepdims=True)
    z = o - m
    o_ref[...] = z - jnp.log(jnp.sum(jnp.exp(z), axis=1, keepdims=True))


def _stage5(acc2, b2):
    blk = 640
    return pl.pallas_call(
        _stage5_body,
        grid=(NPAD // blk,),
        in_specs=[
            pl.BlockSpec((NC, blk, 16), lambda i: (0, i, 0)),
            pl.BlockSpec((1, 2), lambda i: (0, 0)),
        ],
        out_specs=pl.BlockSpec((blk, 2), lambda i: (i, 0)),
        out_shape=jax.ShapeDtypeStruct((NPAD, 2), jnp.float32),
    )(acc2, b2)


@jax.jit
def kernel(x, adjacency, W1, att_src1, att_dst1, b1, W2, att_src2,
           att_dst2, b2):
    t = jnp.arange(64)
    perm = (t % 8) * 8 + t // 8
    w1t = W1[:, perm]
    b1t = b1[perm].reshape(1, 64)
    wa_src = (W1.reshape(D_IN, 8, 8) * att_src1[None]).sum(-1)
    wa_dst = (W1.reshape(D_IN, 8, 8) * att_dst1[None]).sum(-1)
    ws = jnp.concatenate([wa_src, wa_src], axis=1)
    wd = jnp.concatenate([wa_dst, wa_dst], axis=1)
    w2p = W2[perm, :]
    va = w2p @ att_src2[0]
    vb = w2p @ att_dst2[0]
    p2c = jnp.concatenate([w2p, va[:, None], vb[:, None]], axis=1)
    b2r = b2.reshape(1, 2)

    src2d = adjacency[0].reshape(NROWS, ROW)
    dst2d = adjacency[1].reshape(NROWS, ROW)

    h, a_s, a_d = _stage1(x, w1t, ws, wd)
    acch, accp = _edge1(h, a_s, a_d, src2d, dst2d)
    t2c = _stage3(acch, accp, b1t, p2c)
    acc2 = _edge2(t2c.reshape(-1), src2d, dst2d)
    return _stage5(acc2.reshape(NC, NPAD, 16), b2r)[:N_NODES]
